# baseline (speedup 1.0000x reference)
import jax
import jax.numpy as jnp
from jax.experimental import pallas as pl
from jax.experimental.pallas import tpu as pltpu


def _prelu(h, a):
    return jnp.maximum(h, 0.0) + a * jnp.minimum(h, 0.0)


def _net_block(x_ref,
               ew1_ref, ew2_ref, ew3_ref, ew4_ref,
               eb1_ref, eb2_ref, eb3_ref, eb4_ref,
               dw1_ref, dw2_ref, dw3_ref, dw4_ref,
               db1_ref, db2_ref, db3_ref, db4_ref,
               pw_ref, pb_ref, al_ref,
               z_ref, f_ref, r_ref):
    x = x_ref[0]
    al = al_ref[0, 0]

    def dense(h, w_ref, b_ref):
        return (jnp.dot(h.astype(jnp.bfloat16), w_ref[0],
                        preferred_element_type=jnp.float32)
                + b_ref[0])

    h = _prelu(dense(x, ew1_ref, eb1_ref), al[0])
    h = _prelu(dense(h, ew2_ref, eb2_ref), al[1])
    h = _prelu(dense(h, ew3_ref, eb3_ref), al[2])
    z = dense(h, ew4_ref, eb4_ref)

    g = _prelu(dense(z, dw1_ref, db1_ref), al[3])
    g = _prelu(dense(g, dw2_ref, db2_ref), al[4])
    g = _prelu(dense(g, dw3_ref, db3_ref), al[5])
    r = dense(g, dw4_ref, db4_ref)

    f = dense(z, pw_ref, pb_ref)

    z_ref[0] = z
    f_ref[0] = f
    r_ref[0] = r


_TILE_CANDIDATES = (1000, 400, 256, 200, 128, 80, 64, 40, 32, 16, 8)


def kernel(xs, enc_params, dec_params, proj_params):
    view, n, din = xs.shape
    nlayers = len(enc_params[0])

    enc_w = [jnp.stack([p[l][0] for p in enc_params]).astype(jnp.bfloat16)
             for l in range(nlayers)]
    enc_b = [jnp.stack([p[l][1] for p in enc_params])[:, None, :]
             for l in range(nlayers)]
    dec_w = [jnp.stack([p[l][0] for p in dec_params]).astype(jnp.bfloat16)
             for l in range(nlayers)]
    dec_b = [jnp.stack([p[l][1] for p in dec_params])[:, None, :]
             for l in range(nlayers)]
    pw = jnp.stack([p[0] for p in proj_params]).astype(jnp.bfloat16)
    pb = jnp.stack([p[1] for p in proj_params])[:, None, :]
    alphas = jnp.stack([
        jnp.concatenate([e[l][2] for l in range(nlayers - 1)]
                        + [d[l][2] for l in range(nlayers - 1)])
        for e, d in zip(enc_params, dec_params)
    ])[:, None, :]

    tile = next(t for t in _TILE_CANDIDATES if n % t == 0)

    feat = enc_w[-1].shape[-1]
    high = pw.shape[-1]
    out_shape = (
        jax.ShapeDtypeStruct((view, n, feat), xs.dtype),
        jax.ShapeDtypeStruct((view, n, high), xs.dtype),
        jax.ShapeDtypeStruct((view, n, din), xs.dtype),
    )

    def wspec(arr):
        return pl.BlockSpec((1,) + arr.shape[1:], lambda v, i: (v, 0, 0))

    def rowspec(d):
        return pl.BlockSpec((1, tile, d), lambda v, i: (v, i, 0))

    in_specs = ([rowspec(din)]
                + [wspec(w) for w in enc_w] + [wspec(b) for b in enc_b]
                + [wspec(w) for w in dec_w] + [wspec(b) for b in dec_b]
                + [wspec(pw), wspec(pb), wspec(alphas)])
    out_specs = (rowspec(feat), rowspec(high), rowspec(din))

    xs_lo = xs.astype(jnp.bfloat16)
    return pl.pallas_call(
        _net_block,
        grid=(view, n // tile),
        in_specs=in_specs,
        out_specs=out_specs,
        out_shape=out_shape,
        compiler_params=pltpu.CompilerParams(
            dimension_semantics=("arbitrary", "arbitrary"),
            vmem_limit_bytes=100 * 1024 * 1024,
        ),
    )(xs_lo, *enc_w, *enc_b, *dec_w, *dec_b, pw, pb, alphas)


# speedup vs baseline: 1.0228x; 1.0228x over previous
import jax
import jax.numpy as jnp
from jax.experimental import pallas as pl
from jax.experimental.pallas import tpu as pltpu


def _prelu(h, a):
    return jnp.maximum(h, 0.0) + a * jnp.minimum(h, 0.0)


def _net_block(x_ref,
               ew1_ref, ew2_ref, ew3_ref, ew4_ref,
               eb1_ref, eb2_ref, eb3_ref, eb4_ref,
               dw1_ref, dw2_ref, dw3_ref, dw4_ref,
               db1_ref, db2_ref, db3_ref, db4_ref,
               pw_ref, pb_ref, al_ref,
               z_ref, f_ref, r_ref):
    x = x_ref[0]
    al = al_ref[0, 0]

    def dense(h, w_ref, b_ref):
        return (jnp.dot(h, w_ref[0], preferred_element_type=jnp.float32)
                + b_ref[0])

    h = _prelu(dense(x, ew1_ref, eb1_ref), al[0])
    h = _prelu(dense(h, ew2_ref, eb2_ref), al[1])
    h = _prelu(dense(h, ew3_ref, eb3_ref), al[2])
    z = dense(h, ew4_ref, eb4_ref)

    g = _prelu(dense(z, dw1_ref, db1_ref), al[3])
    g = _prelu(dense(g, dw2_ref, db2_ref), al[4])
    g = _prelu(dense(g, dw3_ref, db3_ref), al[5])
    r = dense(g, dw4_ref, db4_ref)

    f = dense(z, pw_ref, pb_ref)

    z_ref[0] = z
    f_ref[0] = f
    r_ref[0] = r


_TILE_CANDIDATES = (1000, 400, 256, 200, 128, 80, 64, 40, 32, 16, 8)


def kernel(xs, enc_params, dec_params, proj_params):
    view, n, din = xs.shape
    nlayers = len(enc_params[0])

    enc_w = [jnp.stack([p[l][0] for p in enc_params]) for l in range(nlayers)]
    enc_b = [jnp.stack([p[l][1] for p in enc_params])[:, None, :]
             for l in range(nlayers)]
    dec_w = [jnp.stack([p[l][0] for p in dec_params]) for l in range(nlayers)]
    dec_b = [jnp.stack([p[l][1] for p in dec_params])[:, None, :]
             for l in range(nlayers)]
    pw = jnp.stack([p[0] for p in proj_params])
    pb = jnp.stack([p[1] for p in proj_params])[:, None, :]
    alphas = jnp.stack([
        jnp.concatenate([e[l][2] for l in range(nlayers - 1)]
                        + [d[l][2] for l in range(nlayers - 1)])
        for e, d in zip(enc_params, dec_params)
    ])[:, None, :]

    tile = next(t for t in _TILE_CANDIDATES if n % t == 0)

    feat = enc_w[-1].shape[-1]
    high = pw.shape[-1]
    out_shape = (
        jax.ShapeDtypeStruct((view, n, feat), xs.dtype),
        jax.ShapeDtypeStruct((view, n, high), xs.dtype),
        jax.ShapeDtypeStruct((view, n, din), xs.dtype),
    )

    def wspec(arr):
        return pl.BlockSpec((1,) + arr.shape[1:], lambda v, i: (v, 0, 0))

    def rowspec(d):
        return pl.BlockSpec((1, tile, d), lambda v, i: (v, i, 0))

    in_specs = ([rowspec(din)]
                + [wspec(w) for w in enc_w] + [wspec(b) for b in enc_b]
                + [wspec(w) for w in dec_w] + [wspec(b) for b in dec_b]
                + [wspec(pw), wspec(pb), wspec(alphas)])
    out_specs = (rowspec(feat), rowspec(high), rowspec(din))

    return pl.pallas_call(
        _net_block,
        grid=(view, n // tile),
        in_specs=in_specs,
        out_specs=out_specs,
        out_shape=out_shape,
        compiler_params=pltpu.CompilerParams(
            dimension_semantics=("arbitrary", "arbitrary"),
            vmem_limit_bytes=100 * 1024 * 1024,
        ),
    )(xs, *enc_w, *enc_b, *dec_w, *dec_b, pw, pb, alphas)


# speedup vs baseline: 1.1071x; 1.0824x over previous
import jax
import jax.numpy as jnp
from jax.experimental import pallas as pl
from jax.experimental.pallas import tpu as pltpu


def _prelu(h, a):
    return jnp.maximum(h, 0.0) + a * jnp.minimum(h, 0.0)


def _dense(h, w_ref, b_ref):
    return jnp.dot(h, w_ref[...], preferred_element_type=jnp.float32) + b_ref[...]


def _net_block(*refs):
    x_ref = refs[0]
    al_ref = refs[1]
    z_ref, f_ref, r_ref = refs[-3:]
    nview = x_ref.shape[0]
    per = (len(refs) - 5) // nview
    for v in range(nview):
        (ew1, eb1, ew2, eb2, ew3, eb3, ew4, eb4,
         dw1, db1, dw2, db2, dw3, db3, dw4, db4,
         pw, pb) = refs[2 + v * per: 2 + (v + 1) * per]
        x = x_ref[v]
        al = al_ref[v, 0]

        h = _prelu(_dense(x, ew1, eb1), al[0])
        h = _prelu(_dense(h, ew2, eb2), al[1])
        h = _prelu(_dense(h, ew3, eb3), al[2])
        z = _dense(h, ew4, eb4)

        g = _prelu(_dense(z, dw1, db1), al[3])
        g = _prelu(_dense(g, dw2, db2), al[4])
        g = _prelu(_dense(g, dw3, db3), al[5])
        r = _dense(g, dw4, db4)

        f = _dense(z, pw, pb)

        z_ref[v] = z
        f_ref[v] = f
        r_ref[v] = r


_TILE_CANDIDATES = (400, 256, 200, 128, 80, 64, 40, 32, 16, 8)


def kernel(xs, enc_params, dec_params, proj_params):
    view, n, din = xs.shape
    nlayers = len(enc_params[0])
    tile = next(t for t in _TILE_CANDIDATES if n % t == 0)

    alphas = jnp.stack([
        jnp.concatenate([e[l][2] for l in range(nlayers - 1)]
                        + [d[l][2] for l in range(nlayers - 1)])
        for e, d in zip(enc_params, dec_params)
    ])[:, None, :]

    def const_spec(arr):
        shape = arr.shape
        return pl.BlockSpec(shape, lambda i: (0,) * len(shape))

    operands = []
    in_specs = [pl.BlockSpec((view, tile, din), lambda i: (0, i, 0)),
                const_spec(alphas)]
    per_view = []
    for v in range(view):
        ops = []
        for (w, b, _a) in enc_params[v]:
            ops += [w, b.reshape(1, -1)]
        for (w, b, _a) in dec_params[v]:
            ops += [w, b.reshape(1, -1)]
        pw, pb = proj_params[v]
        ops += [pw, pb.reshape(1, -1)]
        per_view.append(ops)
    for ops in per_view:
        operands += ops
        in_specs += [const_spec(o) for o in ops]

    feat = enc_params[0][-1][0].shape[-1]
    high = proj_params[0][0].shape[-1]
    out_shape = (
        jax.ShapeDtypeStruct((view, n, feat), xs.dtype),
        jax.ShapeDtypeStruct((view, n, high), xs.dtype),
        jax.ShapeDtypeStruct((view, n, din), xs.dtype),
    )
    out_specs = (
        pl.BlockSpec((view, tile, feat), lambda i: (0, i, 0)),
        pl.BlockSpec((view, tile, high), lambda i: (0, i, 0)),
        pl.BlockSpec((view, tile, din), lambda i: (0, i, 0)),
    )

    return pl.pallas_call(
        _net_block,
        grid=(n // tile,),
        in_specs=in_specs,
        out_specs=out_specs,
        out_shape=out_shape,
        compiler_params=pltpu.CompilerParams(
            dimension_semantics=("arbitrary",),
            vmem_limit_bytes=100 * 1024 * 1024,
        ),
    )(xs, alphas, *operands)
